# baseline (speedup 1.0000x reference)
import functools

import jax
import jax.numpy as jnp
from jax import lax
from jax.experimental import pallas as pl
from jax.experimental.pallas import tpu as pltpu
from jax.experimental.pallas import tpu_sc as plsc

N = 10000
E = 320000
NC, NS = 2, 16
CHUNK = 128
K2 = 158
KH = K2 // 2
EPAD = NS * K2 * CHUNK
RPT = 640
NPT = N // NS
NPAD = NS * RPT
BM = 5000


def _sc_mesh():
    return plsc.VectorSubcoreMesh(core_axis_name="c", subcore_axis_name="s")


_SC_PARAMS = pltpu.CompilerParams(use_tc_tiling_on_sc=False)
_SC_PARAMS_NOLAYOUT = pltpu.CompilerParams(
    use_tc_tiling_on_sc=False, needs_layout_passes=False)


def _sc_degree(dst3, zrows):
    NR = NPAD // 16
    RR = RPT // 16

    @functools.partial(
        pl.kernel,
        out_type=jax.ShapeDtypeStruct((NC, NR, 16), jnp.float32),
        mesh=_sc_mesh(),
        scratch_types=[
            pltpu.VMEM((KH, CHUNK), jnp.int32),
            pltpu.VMEM((NR, 16), jnp.float32),
            pltpu.VMEM((NR,), jnp.int32),
            pltpu.VMEM_SHARED((NR, 16), jnp.float32),
        ],
        compiler_params=_SC_PARAMS_NOLAYOUT,
    )
    def k(dst_h, z_h, out_h, didx, hist, ridx, acc):
        cid = lax.axis_index("c")
        sid = lax.axis_index("s")

        pltpu.sync_copy(z_h, acc.at[pl.ds(sid * RR, RR)])
        pltpu.sync_copy(dst_h.at[sid, pl.ds(cid * KH, KH)], didx)

        @pl.loop(0, NR)
        def _zero(i):
            hist[i] = jnp.zeros((16,), jnp.float32)

        @pl.loop(0, NR // 16)
        def _iota(i):
            ridx[pl.ds(i * 16, 16)] = lax.iota(jnp.int32, 16) + i * 16

        ones = jnp.full((16,), 1.0, jnp.float32)
        m15 = jnp.full((16,), 15, jnp.int32)

        @pl.loop(0, KH)
        def _body(j):
            @pl.loop(0, CHUNK // 16)
            def _inner(i):
                idx = didx[j, pl.ds(i * 16, 16)]
                hi = lax.shift_right_logical(idx, jnp.full((16,), 4, jnp.int32))
                lo = lax.bitwise_and(idx, m15)
                plsc.addupdate_scatter(hist, [hi, lo], ones)

        plsc.subcore_barrier()
        pltpu.sync_copy(hist, acc.at[ridx], add=True)
        plsc.subcore_barrier()
        pltpu.sync_copy(acc.at[pl.ds(sid * RR, RR)],
                        out_h.at[cid, pl.ds(sid * RR, RR)])

    return k(dst3, zrows)


def _sc_propagate_cols(table, src3, dst3, zrows):
    F2 = table.shape[2]

    @functools.partial(
        pl.kernel,
        out_type=jax.ShapeDtypeStruct((NC, NPAD, F2), jnp.float32),
        mesh=_sc_mesh(),
        scratch_types=[
            pltpu.VMEM((K2, CHUNK), jnp.int32),
            pltpu.VMEM((K2, CHUNK), jnp.int32),
            pltpu.VMEM((CHUNK, F2), jnp.float32),
            pltpu.VMEM((CHUNK, F2), jnp.float32),
            pltpu.VMEM_SHARED((N, F2), jnp.float32),
            pltpu.VMEM_SHARED((NPAD, F2), jnp.float32),
            pltpu.SemaphoreType.DMA,
            pltpu.SemaphoreType.DMA,
            pltpu.SemaphoreType.DMA,
            pltpu.SemaphoreType.DMA,
        ],
        compiler_params=_SC_PARAMS,
    )
    def k(table_h, src_h, dst_h, z_h, out_h, sidx, didx, rows0, rows1,
          tbl, acc, sg0, sg1, ss0, ss1):
        cid = lax.axis_index("c")
        sid = lax.axis_index("s")

        pltpu.sync_copy(table_h.at[cid, pl.ds(sid * NPT, NPT)],
                        tbl.at[pl.ds(sid * NPT, NPT)])
        pltpu.sync_copy(z_h, acc.at[pl.ds(sid * RPT, RPT)])
        pltpu.sync_copy(src_h.at[sid], sidx)
        pltpu.sync_copy(dst_h.at[sid], didx)
        plsc.subcore_barrier()

        @pl.loop(0, K2 // 2)
        def _body(r):
            j = r * 2
            g0 = pltpu.async_copy(tbl.at[sidx.at[j]], rows0, sg0)
            g1 = pltpu.async_copy(tbl.at[sidx.at[j + 1]], rows1, sg1)
            g0.wait()
            s0 = pltpu.async_copy(rows0, acc.at[didx.at[j]], ss0, add=True)
            g1.wait()
            s1 = pltpu.async_copy(rows1, acc.at[didx.at[j + 1]], ss1, add=True)
            s0.wait()
            s1.wait()

        plsc.subcore_barrier()
        pltpu.sync_copy(acc.at[pl.ds(sid * RPT, RPT)],
                        out_h.at[cid, pl.ds(sid * RPT, RPT)])

    return k(table, src3, dst3, zrows)


def _sc_propagate_edges(table, src3, dst3, zrows):
    F = table.shape[1]

    @functools.partial(
        pl.kernel,
        out_type=jax.ShapeDtypeStruct((NC, NPAD, F), jnp.float32),
        mesh=_sc_mesh(),
        scratch_types=[
            pltpu.VMEM((KH, CHUNK), jnp.int32),
            pltpu.VMEM((KH, CHUNK), jnp.int32),
            pltpu.VMEM((CHUNK, F), jnp.float32),
            pltpu.VMEM((CHUNK, F), jnp.float32),
            pltpu.VMEM_SHARED((N, F), jnp.float32),
            pltpu.VMEM_SHARED((NPAD, F), jnp.float32),
            pltpu.SemaphoreType.DMA,
            pltpu.SemaphoreType.DMA,
            pltpu.SemaphoreType.DMA,
            pltpu.SemaphoreType.DMA,
        ],
        compiler_params=_SC_PARAMS,
    )
    def k(table_h, src_h, dst_h, z_h, out_h, sidx, didx, rows0, rows1,
          tbl, acc, sg0, sg1, ss0, ss1):
        cid = lax.axis_index("c")
        sid = lax.axis_index("s")

        pltpu.sync_copy(table_h.at[pl.ds(sid * NPT, NPT)],
                        tbl.at[pl.ds(sid * NPT, NPT)])
        pltpu.sync_copy(z_h, acc.at[pl.ds(sid * RPT, RPT)])
        pltpu.sync_copy(src_h.at[sid, pl.ds(cid * KH, KH)], sidx)
        pltpu.sync_copy(dst_h.at[sid, pl.ds(cid * KH, KH)], didx)
        plsc.subcore_barrier()

        @pl.loop(0, KH // 2)
        def _body(r):
            j = r * 2
            g0 = pltpu.async_copy(tbl.at[sidx.at[j]], rows0, sg0)
            g1 = pltpu.async_copy(tbl.at[sidx.at[j + 1]], rows1, sg1)
            g0.wait()
            s0 = pltpu.async_copy(rows0, acc.at[didx.at[j]], ss0, add=True)
            g1.wait()
            s1 = pltpu.async_copy(rows1, acc.at[didx.at[j + 1]], ss1, add=True)
            s0.wait()
            s1.wait()

        pltpu.sync_copy(tbl.at[sidx.at[KH - 1]], rows0)
        pltpu.sync_copy(rows0, acc.at[didx.at[KH - 1]], add=True)

        plsc.subcore_barrier()
        pltpu.sync_copy(acc.at[pl.ds(sid * RPT, RPT)],
                        out_h.at[cid, pl.ds(sid * RPT, RPT)])

    return k(table, src3, dst3, zrows)


def _tc_first(x, W1, degP):
    F = W1.shape[1]
    F2 = F // 2

    def body(dg_ref, x_ref, w_ref, dinv_ref, ts_ref):
        deg = dg_ref[0] + dg_ref[1] + 1.0
        dinv = lax.rsqrt(deg)
        dinv_ref[...] = dinv
        h = jnp.dot(x_ref[...], w_ref[...], preferred_element_type=jnp.float32)
        h = h * dinv
        ts_ref[0] = h[:, :F2]
        ts_ref[1] = h[:, F2:]

    return pl.pallas_call(
        body,
        grid=(N // BM,),
        in_specs=[
            pl.BlockSpec((NC, BM, 1), lambda i: (0, i, 0)),
            pl.BlockSpec((BM, x.shape[1]), lambda i: (i, 0)),
            pl.BlockSpec(W1.shape, lambda i: (0, 0)),
        ],
        out_specs=[
            pl.BlockSpec((BM, 1), lambda i: (i, 0)),
            pl.BlockSpec((NC, BM, F2), lambda i: (0, i, 0)),
        ],
        out_shape=[
            jax.ShapeDtypeStruct((N, 1), jnp.float32),
            jax.ShapeDtypeStruct((NC, N, F2), jnp.float32),
        ],
    )(degP, x, W1)


def _tc_mid(P, p_cols, ts, ts_cols, dinv, b, Wn, out_cols):
    Fp = (P.shape[2] * 2) if p_cols else P.shape[2]
    Fn = Wn.shape[1]

    def body(p_ref, ts_ref, dinv_ref, b_ref, w_ref, o_ref):
        dv = dinv_ref[...]
        if p_cols:
            p = jnp.concatenate([p_ref[0], p_ref[1]], axis=1)
        else:
            p = p_ref[0] + p_ref[1]
        if ts_cols:
            t = jnp.concatenate([ts_ref[0], ts_ref[1]], axis=1)
        else:
            t = ts_ref[...]
        z = dv * (p + t) + b_ref[...]
        h = jnp.maximum(z, 0.0)
        o = jnp.dot(h, w_ref[...], preferred_element_type=jnp.float32) * dv
        if out_cols:
            o_ref[0] = o[:, :Fn // 2]
            o_ref[1] = o[:, Fn // 2:]
        else:
            o_ref[...] = o

    ts_spec = (pl.BlockSpec((NC, BM, Fp // 2), lambda i: (0, i, 0)) if ts_cols
               else pl.BlockSpec((BM, Fp), lambda i: (i, 0)))
    if out_cols:
        out_spec = pl.BlockSpec((NC, BM, Fn // 2), lambda i: (0, i, 0))
        out_shape = jax.ShapeDtypeStruct((NC, N, Fn // 2), jnp.float32)
    else:
        out_spec = pl.BlockSpec((BM, Fn), lambda i: (i, 0))
        out_shape = jax.ShapeDtypeStruct((N, Fn), jnp.float32)

    return pl.pallas_call(
        body,
        grid=(N // BM,),
        in_specs=[
            pl.BlockSpec((NC, BM, P.shape[2]), lambda i: (0, i, 0)),
            ts_spec,
            pl.BlockSpec((BM, 1), lambda i: (i, 0)),
            pl.BlockSpec((1, Fp), lambda i: (0, 0)),
            pl.BlockSpec((Fp, Fn), lambda i: (0, 0)),
        ],
        out_specs=out_spec,
        out_shape=out_shape,
    )(P, ts, dinv, b, Wn)


def _tc_scale(P, ts, dinv, b):
    Fp = ts.shape[1]

    def body(p_ref, ts_ref, dinv_ref, b_ref, o_ref):
        dv = dinv_ref[...]
        z = dv * (p_ref[0] + p_ref[1] + ts_ref[...]) + b_ref[...]
        o_ref[...] = jnp.maximum(z, 0.0) * dv

    return pl.pallas_call(
        body,
        grid=(N // BM,),
        in_specs=[
            pl.BlockSpec((NC, BM, Fp), lambda i: (0, i, 0)),
            pl.BlockSpec((BM, Fp), lambda i: (i, 0)),
            pl.BlockSpec((BM, 1), lambda i: (i, 0)),
            pl.BlockSpec((1, Fp), lambda i: (0, 0)),
        ],
        out_specs=pl.BlockSpec((BM, Fp), lambda i: (i, 0)),
        out_shape=jax.ShapeDtypeStruct((N, Fp), jnp.float32),
    )(P, ts, dinv, b)


def _tc_final(P, ts, dinv, W6, b):
    Fp = ts.shape[1]
    Fn = W6.shape[1]

    def body(p_ref, ts_ref, dinv_ref, b_ref, w_ref, o_ref):
        q = dinv_ref[...] * (p_ref[0] + p_ref[1] + ts_ref[...])
        z = jnp.dot(q, w_ref[...], preferred_element_type=jnp.float32) + b_ref[...]
        m = jnp.max(z, axis=1, keepdims=True)
        o_ref[...] = z - m - jnp.log(jnp.sum(jnp.exp(z - m), axis=1, keepdims=True))

    return pl.pallas_call(
        body,
        grid=(N // BM,),
        in_specs=[
            pl.BlockSpec((NC, BM, Fp), lambda i: (0, i, 0)),
            pl.BlockSpec((BM, Fp), lambda i: (i, 0)),
            pl.BlockSpec((BM, 1), lambda i: (i, 0)),
            pl.BlockSpec((1, Fn), lambda i: (0, 0)),
            pl.BlockSpec((Fp, Fn), lambda i: (0, 0)),
        ],
        out_specs=pl.BlockSpec((BM, Fn), lambda i: (i, 0)),
        out_shape=jax.ShapeDtypeStruct((N, Fn), jnp.float32),
    )(P, ts, dinv, b, W6)


def kernel(x, edge_index, W1, b1, W2, b2, W3, b3, W4, b4, W5, b5, W6, b6):
    pad = EPAD - E
    src3 = jnp.concatenate(
        [edge_index[0], jnp.zeros((pad,), jnp.int32)]).reshape(NS, K2, CHUNK)
    dst3 = jnp.concatenate(
        [edge_index[1], jnp.full((pad,), N, jnp.int32)]).reshape(NS, K2, CHUNK)

    def zr(f):
        return jnp.zeros((RPT, f), jnp.float32)

    degP = _sc_degree(dst3, jnp.zeros((RPT // 16, 16), jnp.float32))
    dinv, ts = _tc_first(x, W1, degP.reshape(NC, NPAD, 1))

    P = _sc_propagate_cols(ts, src3, dst3, zr(48))
    ts = _tc_mid(P, True, ts, True, dinv, b1.reshape(1, -1), W2, True)
    P = _sc_propagate_cols(ts, src3, dst3, zr(32))
    ts = _tc_mid(P, True, ts, True, dinv, b2.reshape(1, -1), W3, False)
    P = _sc_propagate_edges(ts, src3, dst3, zr(48))
    ts = _tc_mid(P, False, ts, False, dinv, b3.reshape(1, -1), W4, False)
    P = _sc_propagate_edges(ts, src3, dst3, zr(32))
    ts = _tc_mid(P, False, ts, False, dinv, b4.reshape(1, -1), W5, False)
    P = _sc_propagate_edges(ts, src3, dst3, zr(16))
    ts = _tc_scale(P, ts, dinv, b5.reshape(1, -1))
    P = _sc_propagate_edges(ts, src3, dst3, zr(16))
    return _tc_final(P, ts, dinv, W6, b6.reshape(1, -1))


# speedup vs baseline: 1.0072x; 1.0072x over previous
import functools

import jax
import jax.numpy as jnp
from jax import lax
from jax.experimental import pallas as pl
from jax.experimental.pallas import tpu as pltpu
from jax.experimental.pallas import tpu_sc as plsc

N = 10000
E = 320000
NC, NS = 2, 16
CHUNK = 128
K2 = 158
KH = K2 // 2
EPAD = NS * K2 * CHUNK
RPT = 640
NPT = N // NS
NPAD = NS * RPT
BM = 5000


def _sc_mesh():
    return plsc.VectorSubcoreMesh(core_axis_name="c", subcore_axis_name="s")


_SC_PARAMS = pltpu.CompilerParams(use_tc_tiling_on_sc=False)
_SC_PARAMS_NOLAYOUT = pltpu.CompilerParams(
    use_tc_tiling_on_sc=False, needs_layout_passes=False)


def _sc_degree(dst3, zrows):
    NR = NPAD // 16
    RR = RPT // 16

    @functools.partial(
        pl.kernel,
        out_type=jax.ShapeDtypeStruct((NC, NR, 16), jnp.float32),
        mesh=_sc_mesh(),
        scratch_types=[
            pltpu.VMEM((KH, CHUNK), jnp.int32),
            pltpu.VMEM((NR, 16), jnp.float32),
            pltpu.VMEM((NR,), jnp.int32),
            pltpu.VMEM_SHARED((NR, 16), jnp.float32),
        ],
        compiler_params=_SC_PARAMS_NOLAYOUT,
    )
    def k(dst_h, z_h, out_h, didx, hist, ridx, acc):
        cid = lax.axis_index("c")
        sid = lax.axis_index("s")

        pltpu.sync_copy(z_h, acc.at[pl.ds(sid * RR, RR)])
        pltpu.sync_copy(dst_h.at[sid, pl.ds(cid * KH, KH)], didx)

        @pl.loop(0, NR)
        def _zero(i):
            hist[i] = jnp.zeros((16,), jnp.float32)

        @pl.loop(0, NR // 16)
        def _iota(i):
            ridx[pl.ds(i * 16, 16)] = lax.iota(jnp.int32, 16) + i * 16

        ones = jnp.full((16,), 1.0, jnp.float32)
        m15 = jnp.full((16,), 15, jnp.int32)

        @pl.loop(0, KH)
        def _body(j):
            @pl.loop(0, CHUNK // 16)
            def _inner(i):
                idx = didx[j, pl.ds(i * 16, 16)]
                hi = lax.shift_right_logical(idx, jnp.full((16,), 4, jnp.int32))
                lo = lax.bitwise_and(idx, m15)
                plsc.addupdate_scatter(hist, [hi, lo], ones)

        plsc.subcore_barrier()
        pltpu.sync_copy(hist, acc.at[ridx], add=True)
        plsc.subcore_barrier()
        pltpu.sync_copy(acc.at[pl.ds(sid * RR, RR)],
                        out_h.at[cid, pl.ds(sid * RR, RR)])

    return k(dst3, zrows)


def _sc_propagate_cols(table, src3, dst3, zrows):
    F2 = table.shape[2]

    @functools.partial(
        pl.kernel,
        out_type=jax.ShapeDtypeStruct((NC, NPAD, F2), jnp.float32),
        mesh=_sc_mesh(),
        scratch_types=[
            pltpu.VMEM((K2, CHUNK), jnp.int32),
            pltpu.VMEM((K2, CHUNK), jnp.int32),
            pltpu.VMEM((CHUNK, F2), jnp.float32),
            pltpu.VMEM_SHARED((N, F2), jnp.float32),
            pltpu.VMEM_SHARED((NPAD, F2), jnp.float32),
        ],
        compiler_params=_SC_PARAMS,
    )
    def k(table_h, src_h, dst_h, z_h, out_h, sidx, didx, rows, tbl, acc):
        cid = lax.axis_index("c")
        sid = lax.axis_index("s")

        pltpu.sync_copy(table_h.at[cid, pl.ds(sid * NPT, NPT)],
                        tbl.at[pl.ds(sid * NPT, NPT)])
        pltpu.sync_copy(z_h, acc.at[pl.ds(sid * RPT, RPT)])
        pltpu.sync_copy(src_h.at[sid], sidx)
        pltpu.sync_copy(dst_h.at[sid], didx)
        plsc.subcore_barrier()

        @pl.loop(0, K2)
        def _body(j):
            pltpu.sync_copy(tbl.at[sidx.at[j]], rows)
            pltpu.sync_copy(rows, acc.at[didx.at[j]], add=True)

        plsc.subcore_barrier()
        pltpu.sync_copy(acc.at[pl.ds(sid * RPT, RPT)],
                        out_h.at[cid, pl.ds(sid * RPT, RPT)])

    return k(table, src3, dst3, zrows)


def _sc_propagate_edges(table, src3, dst3, zrows):
    F = table.shape[1]

    @functools.partial(
        pl.kernel,
        out_type=jax.ShapeDtypeStruct((NC, NPAD, F), jnp.float32),
        mesh=_sc_mesh(),
        scratch_types=[
            pltpu.VMEM((KH, CHUNK), jnp.int32),
            pltpu.VMEM((KH, CHUNK), jnp.int32),
            pltpu.VMEM((CHUNK, F), jnp.float32),
            pltpu.VMEM((CHUNK, F), jnp.float32),
            pltpu.VMEM_SHARED((N, F), jnp.float32),
            pltpu.VMEM_SHARED((NPAD, F), jnp.float32),
            pltpu.SemaphoreType.DMA,
            pltpu.SemaphoreType.DMA,
            pltpu.SemaphoreType.DMA,
            pltpu.SemaphoreType.DMA,
        ],
        compiler_params=_SC_PARAMS,
    )
    def k(table_h, src_h, dst_h, z_h, out_h, sidx, didx, rows0, rows1,
          tbl, acc, sg0, sg1, ss0, ss1):
        cid = lax.axis_index("c")
        sid = lax.axis_index("s")

        pltpu.sync_copy(table_h.at[pl.ds(sid * NPT, NPT)],
                        tbl.at[pl.ds(sid * NPT, NPT)])
        pltpu.sync_copy(z_h, acc.at[pl.ds(sid * RPT, RPT)])
        pltpu.sync_copy(src_h.at[sid, pl.ds(cid * KH, KH)], sidx)
        pltpu.sync_copy(dst_h.at[sid, pl.ds(cid * KH, KH)], didx)
        plsc.subcore_barrier()

        @pl.loop(0, KH // 2)
        def _body(r):
            j = r * 2
            g0 = pltpu.async_copy(tbl.at[sidx.at[j]], rows0, sg0)
            g1 = pltpu.async_copy(tbl.at[sidx.at[j + 1]], rows1, sg1)
            g0.wait()
            s0 = pltpu.async_copy(rows0, acc.at[didx.at[j]], ss0, add=True)
            g1.wait()
            s1 = pltpu.async_copy(rows1, acc.at[didx.at[j + 1]], ss1, add=True)
            s0.wait()
            s1.wait()

        pltpu.sync_copy(tbl.at[sidx.at[KH - 1]], rows0)
        pltpu.sync_copy(rows0, acc.at[didx.at[KH - 1]], add=True)

        plsc.subcore_barrier()
        pltpu.sync_copy(acc.at[pl.ds(sid * RPT, RPT)],
                        out_h.at[cid, pl.ds(sid * RPT, RPT)])

    return k(table, src3, dst3, zrows)


def _tc_first(x, W1, degP):
    F = W1.shape[1]
    F2 = F // 2

    def body(dg_ref, x_ref, w_ref, dinv_ref, ts_ref):
        deg = dg_ref[0] + dg_ref[1] + 1.0
        dinv = lax.rsqrt(deg)
        dinv_ref[...] = dinv
        h = jnp.dot(x_ref[...], w_ref[...], preferred_element_type=jnp.float32)
        h = h * dinv
        ts_ref[0] = h[:, :F2]
        ts_ref[1] = h[:, F2:]

    return pl.pallas_call(
        body,
        grid=(N // BM,),
        in_specs=[
            pl.BlockSpec((NC, BM, 1), lambda i: (0, i, 0)),
            pl.BlockSpec((BM, x.shape[1]), lambda i: (i, 0)),
            pl.BlockSpec(W1.shape, lambda i: (0, 0)),
        ],
        out_specs=[
            pl.BlockSpec((BM, 1), lambda i: (i, 0)),
            pl.BlockSpec((NC, BM, F2), lambda i: (0, i, 0)),
        ],
        out_shape=[
            jax.ShapeDtypeStruct((N, 1), jnp.float32),
            jax.ShapeDtypeStruct((NC, N, F2), jnp.float32),
        ],
    )(degP, x, W1)


def _tc_mid(P, p_cols, ts, ts_cols, dinv, b, Wn, out_cols):
    Fp = (P.shape[2] * 2) if p_cols else P.shape[2]
    Fn = Wn.shape[1]

    def body(p_ref, ts_ref, dinv_ref, b_ref, w_ref, o_ref):
        dv = dinv_ref[...]
        if p_cols:
            p = jnp.concatenate([p_ref[0], p_ref[1]], axis=1)
        else:
            p = p_ref[0] + p_ref[1]
        if ts_cols:
            t = jnp.concatenate([ts_ref[0], ts_ref[1]], axis=1)
        else:
            t = ts_ref[...]
        z = dv * (p + t) + b_ref[...]
        h = jnp.maximum(z, 0.0)
        o = jnp.dot(h, w_ref[...], preferred_element_type=jnp.float32) * dv
        if out_cols:
            o_ref[0] = o[:, :Fn // 2]
            o_ref[1] = o[:, Fn // 2:]
        else:
            o_ref[...] = o

    ts_spec = (pl.BlockSpec((NC, BM, Fp // 2), lambda i: (0, i, 0)) if ts_cols
               else pl.BlockSpec((BM, Fp), lambda i: (i, 0)))
    if out_cols:
        out_spec = pl.BlockSpec((NC, BM, Fn // 2), lambda i: (0, i, 0))
        out_shape = jax.ShapeDtypeStruct((NC, N, Fn // 2), jnp.float32)
    else:
        out_spec = pl.BlockSpec((BM, Fn), lambda i: (i, 0))
        out_shape = jax.ShapeDtypeStruct((N, Fn), jnp.float32)

    return pl.pallas_call(
        body,
        grid=(N // BM,),
        in_specs=[
            pl.BlockSpec((NC, BM, P.shape[2]), lambda i: (0, i, 0)),
            ts_spec,
            pl.BlockSpec((BM, 1), lambda i: (i, 0)),
            pl.BlockSpec((1, Fp), lambda i: (0, 0)),
            pl.BlockSpec((Fp, Fn), lambda i: (0, 0)),
        ],
        out_specs=out_spec,
        out_shape=out_shape,
    )(P, ts, dinv, b, Wn)


def _tc_scale(P, ts, dinv, b):
    Fp = ts.shape[1]

    def body(p_ref, ts_ref, dinv_ref, b_ref, o_ref):
        dv = dinv_ref[...]
        z = dv * (p_ref[0] + p_ref[1] + ts_ref[...]) + b_ref[...]
        o_ref[...] = jnp.maximum(z, 0.0) * dv

    return pl.pallas_call(
        body,
        grid=(N // BM,),
        in_specs=[
            pl.BlockSpec((NC, BM, Fp), lambda i: (0, i, 0)),
            pl.BlockSpec((BM, Fp), lambda i: (i, 0)),
            pl.BlockSpec((BM, 1), lambda i: (i, 0)),
            pl.BlockSpec((1, Fp), lambda i: (0, 0)),
        ],
        out_specs=pl.BlockSpec((BM, Fp), lambda i: (i, 0)),
        out_shape=jax.ShapeDtypeStruct((N, Fp), jnp.float32),
    )(P, ts, dinv, b)


def _tc_final(P, ts, dinv, W6, b):
    Fp = ts.shape[1]
    Fn = W6.shape[1]

    def body(p_ref, ts_ref, dinv_ref, b_ref, w_ref, o_ref):
        q = dinv_ref[...] * (p_ref[0] + p_ref[1] + ts_ref[...])
        z = jnp.dot(q, w_ref[...], preferred_element_type=jnp.float32) + b_ref[...]
        m = jnp.max(z, axis=1, keepdims=True)
        o_ref[...] = z - m - jnp.log(jnp.sum(jnp.exp(z - m), axis=1, keepdims=True))

    return pl.pallas_call(
        body,
        grid=(N // BM,),
        in_specs=[
            pl.BlockSpec((NC, BM, Fp), lambda i: (0, i, 0)),
            pl.BlockSpec((BM, Fp), lambda i: (i, 0)),
            pl.BlockSpec((BM, 1), lambda i: (i, 0)),
            pl.BlockSpec((1, Fn), lambda i: (0, 0)),
            pl.BlockSpec((Fp, Fn), lambda i: (0, 0)),
        ],
        out_specs=pl.BlockSpec((BM, Fn), lambda i: (i, 0)),
        out_shape=jax.ShapeDtypeStruct((N, Fn), jnp.float32),
    )(P, ts, dinv, b, W6)


def kernel(x, edge_index, W1, b1, W2, b2, W3, b3, W4, b4, W5, b5, W6, b6):
    pad = EPAD - E
    src3 = jnp.concatenate(
        [edge_index[0], jnp.zeros((pad,), jnp.int32)]).reshape(NS, K2, CHUNK)
    dst3 = jnp.concatenate(
        [edge_index[1], jnp.full((pad,), N, jnp.int32)]).reshape(NS, K2, CHUNK)

    def zr(f):
        return jnp.zeros((RPT, f), jnp.float32)

    degP = _sc_degree(dst3, jnp.zeros((RPT // 16, 16), jnp.float32))
    dinv, ts = _tc_first(x, W1, degP.reshape(NC, NPAD, 1))

    P = _sc_propagate_cols(ts, src3, dst3, zr(48))
    ts = _tc_mid(P, True, ts, True, dinv, b1.reshape(1, -1), W2, True)
    P = _sc_propagate_cols(ts, src3, dst3, zr(32))
    ts = _tc_mid(P, True, ts, True, dinv, b2.reshape(1, -1), W3, False)
    P = _sc_propagate_edges(ts, src3, dst3, zr(48))
    ts = _tc_mid(P, False, ts, False, dinv, b3.reshape(1, -1), W4, False)
    P = _sc_propagate_edges(ts, src3, dst3, zr(32))
    ts = _tc_mid(P, False, ts, False, dinv, b4.reshape(1, -1), W5, False)
    P = _sc_propagate_edges(ts, src3, dst3, zr(16))
    ts = _tc_scale(P, ts, dinv, b5.reshape(1, -1))
    P = _sc_propagate_edges(ts, src3, dst3, zr(16))
    return _tc_final(P, ts, dinv, W6, b6.reshape(1, -1))
